# baseline (speedup 1.0000x reference)
import functools

import jax
import jax.numpy as jnp
from jax import lax
from jax.experimental import pallas as pl
from jax.experimental.pallas import tpu as pltpu
from jax.experimental.pallas import tpu_sc as plsc

D_MODEL = 1024
NUM_EXPERTS = 8
SEQ = 2048
NBUF = 3
NEG_INF = float("-inf")


def _scores_body(x_ref, router_W_ref, router_b_ref, scores_ref):
    scores = lax.dot_general(router_W_ref[...], x_ref[...],
                             (((1,), (1,)), ((), ())),
                             preferred_element_type=jnp.float32)
    scores_ref[...] = scores + router_b_ref[...]


def _route_body(scores_hbm, coeff_hbm, sbuf, cbuf):
    info = plsc.get_sparse_core_info()
    nc = info.num_cores
    wid = lax.axis_index("s") * nc + lax.axis_index("c")
    chunk = SEQ // (nc * info.num_subcores)
    base = wid * chunk
    for e in range(NUM_EXPERTS):
        pltpu.sync_copy(scores_hbm.at[pl.ds(e * SEQ + base, chunk)],
                        sbuf.at[pl.ds(e * chunk, chunk)])
    for j in range(chunk // 16):
        s = [sbuf[pl.ds(e * chunk + j * 16, 16)]
             for e in range(NUM_EXPERTS)]
        m0 = s[0]
        for e in range(1, NUM_EXPERTS):
            m0 = jnp.maximum(m0, s[e])
        a0 = jnp.full((16,), NUM_EXPERTS, jnp.int32)
        for e in range(NUM_EXPERTS - 1, -1, -1):
            a0 = jnp.where(s[e] == m0, jnp.int32(e), a0)
        masked = [jnp.where(a0 == e, jnp.float32(NEG_INF), s[e])
                  for e in range(NUM_EXPERTS)]
        m1 = masked[0]
        for e in range(1, NUM_EXPERTS):
            m1 = jnp.maximum(m1, masked[e])
        a1 = jnp.full((16,), NUM_EXPERTS, jnp.int32)
        for e in range(NUM_EXPERTS - 1, -1, -1):
            a1 = jnp.where(masked[e] == m1, jnp.int32(e), a1)
        z = jnp.exp(m1 - m0)
        w0 = 1.0 / (1.0 + z)
        w1 = z * w0
        zero = jnp.zeros((16,), jnp.float32)
        for e in range(NUM_EXPERTS):
            ce = (jnp.where(a0 == e, w0, zero)
                  + jnp.where(a1 == e, w1, zero))
            cbuf[pl.ds(e * chunk + j * 16, 16)] = ce
    for e in range(NUM_EXPERTS):
        pltpu.sync_copy(cbuf.at[pl.ds(e * chunk, chunk)],
                        coeff_hbm.at[pl.ds(e * SEQ + base, chunk)])


def _moe_body(x_ref, shared_W_ref, shared_b_ref, coeff_in_ref, ew_hbm,
              eb_ref, out_ref, xbf_ref, wbuf_ref, sems):
    u = pl.program_id(0)

    @pl.when(u == 0)
    def _():
        for k in range(NBUF):
            pltpu.make_async_copy(ew_hbm.at[k], wbuf_ref.at[k],
                                  sems.at[k]).start()
        xb = x_ref[...]
        xb16 = xb.astype(jnp.bfloat16)
        xbf_ref[...] = xb16
        so = lax.dot_general(xb16, shared_W_ref[...].astype(jnp.bfloat16),
                             (((1,), (1,)), ((), ())),
                             preferred_element_type=jnp.float32)
        bias_mix = lax.dot_general(coeff_in_ref[...], eb_ref[...],
                                   (((0,), (0,)), ((), ())),
                                   preferred_element_type=jnp.float32)
        out_ref[...] = so + shared_b_ref[...] + bias_mix

    @pl.when(u > 0)
    def _():
        e = u - 1
        slot = lax.rem(e, NBUF)
        pltpu.make_async_copy(ew_hbm.at[e], wbuf_ref.at[slot],
                              sems.at[slot]).wait()
        onehot = (lax.broadcasted_iota(jnp.int32, (NUM_EXPERTS, 1), 0)
                  == e).astype(jnp.float32)
        coeff = lax.dot_general(coeff_in_ref[...], onehot,
                                (((0,), (0,)), ((), ())),
                                preferred_element_type=jnp.float32)
        eo = lax.dot_general(xbf_ref[...],
                             wbuf_ref[slot].astype(jnp.bfloat16),
                             (((1,), (1,)), ((), ())),
                             preferred_element_type=jnp.float32)
        out_ref[...] += coeff * eo
        nxt = e + NBUF

        @pl.when(nxt < NUM_EXPERTS)
        def _():
            nslot = lax.rem(nxt, NBUF)
            pltpu.make_async_copy(ew_hbm.at[nxt], wbuf_ref.at[nslot],
                                  sems.at[nslot]).start()


@jax.jit
def kernel(x, shared_W, shared_b, router_W, router_b, expert_W, expert_b):
    B, S, D = x.shape
    x2 = x.reshape(S, D)

    scores = pl.pallas_call(
        _scores_body,
        in_specs=[
            pl.BlockSpec((S, D), lambda: (0, 0)),
            pl.BlockSpec((NUM_EXPERTS, D), lambda: (0, 0)),
            pl.BlockSpec((NUM_EXPERTS, 1), lambda: (0, 0)),
        ],
        out_specs=pl.BlockSpec((NUM_EXPERTS, S), lambda: (0, 0)),
        out_shape=jax.ShapeDtypeStruct((NUM_EXPERTS, S), jnp.float32),
    )(x2, router_W, router_b.reshape(NUM_EXPERTS, 1))

    info = plsc.get_sparse_core_info()
    chunk = S // (info.num_cores * info.num_subcores)
    nflat = chunk * NUM_EXPERTS
    route = pl.kernel(
        _route_body,
        out_type=jax.ShapeDtypeStruct((S * NUM_EXPERTS,), jnp.float32),
        mesh=plsc.VectorSubcoreMesh(core_axis_name="c",
                                    subcore_axis_name="s"),
        scratch_types=[pltpu.VMEM((nflat,), jnp.float32),
                       pltpu.VMEM((nflat,), jnp.float32)],
    )
    coeff = route(scores.reshape(S * NUM_EXPERTS)).reshape(NUM_EXPERTS, S)

    out = pl.pallas_call(
        _moe_body,
        grid=(NUM_EXPERTS + 1,),
        in_specs=[
            pl.BlockSpec((S, D), lambda u: (0, 0)),
            pl.BlockSpec((D, D), lambda u: (0, 0)),
            pl.BlockSpec((1, D), lambda u: (0, 0)),
            pl.BlockSpec((NUM_EXPERTS, S), lambda u: (0, 0)),
            pl.BlockSpec(memory_space=pl.ANY),
            pl.BlockSpec((NUM_EXPERTS, D), lambda u: (0, 0)),
        ],
        out_specs=pl.BlockSpec((S, D), lambda u: (0, 0)),
        out_shape=jax.ShapeDtypeStruct((S, D), jnp.float32),
        scratch_shapes=[pltpu.VMEM((S, D), jnp.bfloat16),
                        pltpu.VMEM((NBUF, D, D), jnp.float32),
                        pltpu.SemaphoreType.DMA((NBUF,))],
    )(x2, shared_W, shared_b.reshape(1, D), coeff, expert_W, expert_b)
    return out.reshape(B, S, D)


# speedup vs baseline: 1.4352x; 1.4352x over previous
import jax
import jax.numpy as jnp
from jax import lax
from jax.experimental import pallas as pl
from jax.experimental.pallas import tpu as pltpu

D_MODEL = 1024
NUM_EXPERTS = 8
SEQ = 2048
CSPLIT = 2
CHALF = D_MODEL // CSPLIT


def _moe_body(x_ref, shared_W_ref, shared_b_ref, router_W_ref,
              router_b_ref, expert_W_ref, expert_b_ref, out_ref,
              coeff_ref, xbf_ref):
    c = pl.program_id(0)
    u = pl.program_id(1)

    @pl.when(u == 0)
    def _():
        @pl.when(c == 0)
        def _():
            xb = x_ref[...]
            xbf_ref[...] = xb.astype(jnp.bfloat16)
            scores = lax.dot_general(xb, router_W_ref[...],
                                     (((1,), (1,)), ((), ())),
                                     preferred_element_type=jnp.float32)
            scores = scores + router_b_ref[...]
            eidx = lax.broadcasted_iota(jnp.int32, scores.shape, 1)
            m0 = jnp.max(scores, axis=-1, keepdims=True)
            a0 = jnp.min(jnp.where(scores == m0, eidx, NUM_EXPERTS),
                         axis=-1, keepdims=True)
            masked = jnp.where(eidx == a0, -jnp.inf, scores)
            m1 = jnp.max(masked, axis=-1, keepdims=True)
            a1 = jnp.min(jnp.where(masked == m1, eidx, NUM_EXPERTS),
                         axis=-1, keepdims=True)
            z = jnp.exp(m1 - m0)
            w0 = 1.0 / (1.0 + z)
            w1 = z * w0
            coeff_ref[...] = (jnp.where(eidx == a0, w0, 0.0)
                              + jnp.where(eidx == a1, w1, 0.0))

        so = lax.dot_general(xbf_ref[...],
                             shared_W_ref[0].astype(jnp.bfloat16),
                             (((1,), (1,)), ((), ())),
                             preferred_element_type=jnp.float32)
        out_ref[...] = so + shared_b_ref[0]

    @pl.when(u > 0)
    def _():
        e = u - 1
        eo = lax.dot_general(xbf_ref[...],
                             expert_W_ref[0, 0].astype(jnp.bfloat16),
                             (((1,), (1,)), ((), ())),
                             preferred_element_type=jnp.float32)
        call = coeff_ref[...]
        lane = lax.broadcasted_iota(jnp.int32, call.shape, 1)
        coeff = jnp.sum(jnp.where(lane == e, call, 0.0), axis=1,
                        keepdims=True)
        out_ref[...] += coeff * (eo + expert_b_ref[0, 0])


@jax.jit
def kernel(x, shared_W, shared_b, router_W, router_b, expert_W, expert_b):
    B, S, D = x.shape
    x2 = x.reshape(S, D)

    def _w_idx(c, u):
        return (jnp.maximum(u - 1, 0), c, 0, 0)

    out = pl.pallas_call(
        _moe_body,
        grid=(CSPLIT, NUM_EXPERTS + 1),
        in_specs=[
            pl.BlockSpec((S, D), lambda c, u: (0, 0)),
            pl.BlockSpec((1, CHALF, D), lambda c, u: (c, 0, 0)),
            pl.BlockSpec((1, 1, CHALF), lambda c, u: (c, 0, 0)),
            pl.BlockSpec((NUM_EXPERTS, D), lambda c, u: (0, 0)),
            pl.BlockSpec((1, NUM_EXPERTS), lambda c, u: (0, 0)),
            pl.BlockSpec((1, 1, CHALF, D), _w_idx),
            pl.BlockSpec((1, 1, 1, CHALF), lambda c, u:
                         (jnp.maximum(u - 1, 0), c, 0, 0)),
        ],
        out_specs=pl.BlockSpec((S, CHALF), lambda c, u: (0, c)),
        out_shape=jax.ShapeDtypeStruct((S, D), jnp.float32),
        scratch_shapes=[pltpu.VMEM((S, NUM_EXPERTS), jnp.float32),
                        pltpu.VMEM((S, D), jnp.bfloat16)],
    )(x2, shared_W.reshape(CSPLIT, CHALF, D),
      shared_b.reshape(CSPLIT, 1, CHALF),
      router_W, router_b.reshape(1, NUM_EXPERTS),
      expert_W.reshape(NUM_EXPERTS, CSPLIT, CHALF, D),
      expert_b.reshape(NUM_EXPERTS, CSPLIT, 1, CHALF))
    return out.reshape(B, S, D)


# speedup vs baseline: 1.5281x; 1.0647x over previous
import jax
import jax.numpy as jnp
from jax import lax
from jax.experimental import pallas as pl
from jax.experimental.pallas import tpu as pltpu

D_MODEL = 1024
NUM_EXPERTS = 8
SEQ = 2048


def _moe_body(x_ref, shared_W_ref, shared_b_ref, router_W_ref,
              router_b_ref, expert_W_ref, expert_b_ref, out_ref, coeff_ref):
    u = pl.program_id(0)

    @pl.when(u == 0)
    def _():
        xb = x_ref[...]
        scores = lax.dot_general(xb, router_W_ref[...],
                                 (((1,), (1,)), ((), ())),
                                 preferred_element_type=jnp.float32)
        scores = scores + router_b_ref[...]
        eidx = lax.broadcasted_iota(jnp.int32, scores.shape, 1)
        m0 = jnp.max(scores, axis=-1, keepdims=True)
        a0 = jnp.min(jnp.where(scores == m0, eidx, NUM_EXPERTS), axis=-1,
                     keepdims=True)
        masked = jnp.where(eidx == a0, -jnp.inf, scores)
        m1 = jnp.max(masked, axis=-1, keepdims=True)
        a1 = jnp.min(jnp.where(masked == m1, eidx, NUM_EXPERTS), axis=-1,
                     keepdims=True)
        z = jnp.exp(m1 - m0)
        w0 = 1.0 / (1.0 + z)
        w1 = z * w0
        coeff_ref[...] = (jnp.where(eidx == a0, w0, 0.0)
                          + jnp.where(eidx == a1, w1, 0.0))
        so = lax.dot_general(xb, shared_W_ref[...], (((1,), (1,)), ((), ())),
                             preferred_element_type=jnp.float32)
        out_ref[...] = so + shared_b_ref[...]

    @pl.when(u > 0)
    def _():
        e = u - 1
        xb = x_ref[...]
        eo = lax.dot_general(xb, expert_W_ref[0], (((1,), (1,)), ((), ())),
                             preferred_element_type=jnp.float32)
        call = coeff_ref[...]
        lane = lax.broadcasted_iota(jnp.int32, call.shape, 1)
        coeff = jnp.sum(jnp.where(lane == e, call, 0.0), axis=1,
                        keepdims=True)
        out_ref[...] += coeff * (eo + expert_b_ref[0])


@jax.jit
def kernel(x, shared_W, shared_b, router_W, router_b, expert_W, expert_b):
    B, S, D = x.shape
    x2 = x.reshape(S, D)

    def _w_idx(u):
        e = jnp.maximum(u - 1, 0)
        return (e, 0, 0)

    out = pl.pallas_call(
        _moe_body,
        grid=(NUM_EXPERTS + 1,),
        in_specs=[
            pl.BlockSpec((S, D), lambda u: (0, 0)),
            pl.BlockSpec((D, D), lambda u: (0, 0)),
            pl.BlockSpec((1, D), lambda u: (0, 0)),
            pl.BlockSpec((NUM_EXPERTS, D), lambda u: (0, 0)),
            pl.BlockSpec((1, NUM_EXPERTS), lambda u: (0, 0)),
            pl.BlockSpec((1, D, D), _w_idx),
            pl.BlockSpec((1, 1, D), _w_idx),
        ],
        out_specs=pl.BlockSpec((S, D), lambda u: (0, 0)),
        out_shape=jax.ShapeDtypeStruct((S, D), jnp.float32),
        scratch_shapes=[pltpu.VMEM((S, NUM_EXPERTS), jnp.float32)],
    )(x2, shared_W, shared_b.reshape(1, D),
      router_W, router_b.reshape(1, NUM_EXPERTS),
      expert_W, expert_b.reshape(NUM_EXPERTS, 1, D))
    return out.reshape(B, S, D)
